# Initial kernel scaffold; baseline (speedup 1.0000x reference)
import jax
import jax.numpy as jnp
from jax.experimental import pallas as pl


def kernel(hidden, done, index, state0, W_write, W_pol, b_pol, W_val, b_val):
    raise NotImplementedError("write your pallas kernel here")


# speedup vs baseline: 10.5952x; 10.5952x over previous
import functools

import jax
import jax.numpy as jnp
from jax import lax
from jax.experimental import pallas as pl
from jax.experimental.pallas import tpu as pltpu, tpu_sc as plsc

T, B, C, H, W = 16, 1024, 32, 32, 32
D_IN = 160
N_ACT = 16
N_OUT = N_ACT + 1
TB = T * B

NC = 2
NS = 16
NW = NC * NS
ROWS = TB * C // NW // 128
B_BLK = 128


def _idx_kernel(idx_ref, out_ref):
    idx = idx_ref[...]
    y = idx[:, 0][:, None]
    x = idx[:, 1][:, None]
    tb = lax.broadcasted_iota(jnp.int32, (TB, 1), 0)
    b = tb % B
    base = b * (C * H * W) + y * W + x
    coff = lax.broadcasted_iota(jnp.int32, (1, C), 1) * (H * W)
    out_ref[...] = base + coff


def _sc_gather_body(table_hbm, idx_hbm, out_hbm, idx_v, rows_v, sem):
    wid = lax.axis_index("s") * NC + lax.axis_index("c")
    pltpu.sync_copy(idx_hbm.at[wid], idx_v)

    def body(j, carry):
        pltpu.async_copy(table_hbm.at[idx_v.at[j]], rows_v.at[j], sem).wait()
        return carry

    lax.fori_loop(0, ROWS, body, 0)
    pltpu.sync_copy(rows_v, out_hbm.at[wid])


def _dense_kernel(hid_ref, done_ref, idx_ref, g_ref, ww_ref, wc_ref, bc_ref,
                  out_ref):
    h = hid_ref[...]
    write = jnp.tanh(
        jnp.dot(h.reshape(T * B_BLK, D_IN), ww_ref[...],
                preferred_element_type=jnp.float32)).reshape(T, B_BLK, C)

    pos = idx_ref[...][:, :, 0] % (H * W)
    m = 1.0 - done_ref[...]

    s_iota = lax.broadcasted_iota(jnp.int32, (T, B_BLK), 0)
    rows = []
    prev = jnp.where(s_iota == 0, 1.0, 0.0)
    rows.append(prev)
    for t in range(1, T):
        prev = m[t][None, :] * prev + jnp.where(s_iota == t, 1.0, 0.0)
        rows.append(prev)
    decay = jnp.stack(rows)
    cum = decay[:, 0, :] * m[0][None, :]

    eq = (pos[:, None, :] == pos[None, :, :])
    tri = (lax.broadcasted_iota(jnp.int32, (T, T, 1), 0)
           >= lax.broadcasted_iota(jnp.int32, (T, T, 1), 1))
    match = jnp.where(eq & tri, decay, 0.0)

    read = cum[:, :, None] * g_ref[...]
    for s in range(T):
        read = read + match[:, s, :, None] * write[s][None, :, :]

    out = jnp.concatenate([write, read], axis=-1)
    res = (jnp.dot(out.reshape(T * B_BLK, 2 * C), wc_ref[...],
                   preferred_element_type=jnp.float32) + bc_ref[...])
    out_ref[...] = res.reshape(T, B_BLK, N_OUT)


def kernel(hidden, done, index, state0, W_write, W_pol, b_pol, W_val, b_val):
    index32 = index.astype(jnp.int32)

    idx_all = pl.pallas_call(
        _idx_kernel,
        out_shape=jax.ShapeDtypeStruct((TB, C), jnp.int32),
    )(index32)

    table = state0.reshape(B * C * H * W)
    gather = pl.kernel(
        _sc_gather_body,
        out_type=jax.ShapeDtypeStruct((NW, ROWS, 128), jnp.float32),
        mesh=plsc.VectorSubcoreMesh(core_axis_name="c", subcore_axis_name="s",
                                    num_cores=NC, num_subcores=NS),
        scratch_types=[
            pltpu.VMEM((ROWS, 128), jnp.int32),
            pltpu.VMEM((ROWS, 128), jnp.float32),
            pltpu.SemaphoreType.DMA,
        ],
    )
    g = gather(table, idx_all.reshape(NW, ROWS, 128))

    w_cat = jnp.concatenate([W_pol, W_val], axis=1)
    b_cat = jnp.concatenate([b_pol, b_val])[None, :]

    out = pl.pallas_call(
        _dense_kernel,
        grid=(B // B_BLK,),
        in_specs=[
            pl.BlockSpec((T, B_BLK, D_IN), lambda i: (0, i, 0)),
            pl.BlockSpec((T, B_BLK), lambda i: (0, i)),
            pl.BlockSpec((T, B_BLK, C), lambda i: (0, i, 0)),
            pl.BlockSpec((T, B_BLK, C), lambda i: (0, i, 0)),
            pl.BlockSpec((D_IN, C), lambda i: (0, 0)),
            pl.BlockSpec((2 * C, N_OUT), lambda i: (0, 0)),
            pl.BlockSpec((1, N_OUT), lambda i: (0, 0)),
        ],
        out_specs=pl.BlockSpec((T, B_BLK, N_OUT), lambda i: (0, i, 0)),
        out_shape=jax.ShapeDtypeStruct((T, B, N_OUT), jnp.float32),
    )(
        hidden.reshape(T, B, D_IN),
        done.reshape(T, B),
        idx_all.reshape(T, B, C),
        g.reshape(T, B, C),
        W_write,
        w_cat,
        b_cat,
    )
    return out.reshape(TB, N_OUT)


# speedup vs baseline: 91.7098x; 8.6558x over previous
import jax
import jax.numpy as jnp
from jax import lax
from jax.experimental import pallas as pl
from jax.experimental.pallas import tpu as pltpu, tpu_sc as plsc

T, B, C, H, W = 16, 1024, 32, 32, 32
D_IN = 160
N_ACT = 16
N_OUT = N_ACT + 1
TB = T * B

NC = 2
NS = 16
NW = NC * NS
CHUNK = 8


def _idx_kernel(idx_ref, out_ref):
    idx = idx_ref[...]
    y = idx[:, 0:1]
    x = idx[:, 1:2]
    tb = lax.broadcasted_iota(jnp.int32, (TB, 1), 0)
    b = tb % B
    off = ((y * 4 + (x >> 3)) * 8192 + (b // 128) * 1024
           + (x & 7) * 128 + (b % 128))
    coff = lax.broadcasted_iota(jnp.int32, (1, C), 1) * (H * W * B)
    out_ref[...] = off + coff


def _sc_gather_body(table_hbm, idx_hbm, out_hbm, idx_v, rows_v, sem):
    wid = lax.axis_index("s") * NC + lax.axis_index("c")
    c8 = wid // 8
    tbb = wid % 8
    rect = (pl.ds(c8 * 8, 8), pl.ds(tbb * 2048, 2048))
    pltpu.sync_copy(idx_hbm.at[rect], idx_v)

    def chunk(jc, carry):
        base = jc * CHUNK
        for k in range(CHUNK):
            j = base + k
            cs = j % 8
            tl = (j // 8) * 128
            pltpu.make_async_copy(
                table_hbm.at[idx_v.at[cs, pl.ds(tl, 128)]],
                rows_v.at[cs, pl.ds(tl, 128)], sem,
            ).start()

        @pl.when(jc > 0)
        def _():
            for k in range(CHUNK):
                j = base - CHUNK + k
                cs = j % 8
                tl = (j // 8) * 128
                pltpu.make_async_copy(
                    table_hbm.at[pl.ds(0, 128)],
                    rows_v.at[cs, pl.ds(tl, 128)], sem,
                ).wait()

        return carry

    n_stream = 8 * 2048 // 128
    lax.fori_loop(0, n_stream // CHUNK, chunk, 0)
    for k in range(CHUNK):
        j = n_stream - CHUNK + k
        cs = j % 8
        tl = (j // 8) * 128
        pltpu.make_async_copy(
            table_hbm.at[pl.ds(0, 128)], rows_v.at[cs, pl.ds(tl, 128)], sem
        ).wait()
    pltpu.sync_copy(rows_v, out_hbm.at[rect])


def _dense_kernel(hT_ref, done_ref, pos_ref, g_ref, ww_ref, wc_ref, bc_ref,
                  out_ref, wv):
    pos = pos_ref[...]
    m = 1.0 - done_ref[...]

    e0 = lax.broadcasted_iota(jnp.int32, (T, B), 0)
    m0 = m[0:1, :]
    prev = jnp.where(e0 == 0, 1.0, 0.0)
    rows, cums = [prev], [m0]
    for t in range(1, T):
        mt = m[t:t + 1, :]
        prev = mt * prev + jnp.where(e0 == t, 1.0, 0.0)
        rows.append(prev)
        cums.append(prev[0:1, :] * m0)

    ww = ww_ref[...]
    w_wr = wc_ref[0:C, :]
    w_rd = wc_ref[C:2 * C, :]
    bc = bc_ref[...][:, None]

    for t in range(T):
        lo, hi = t * B, (t + 1) * B
        w_col = jnp.tanh(
            lax.dot_general(ww, hT_ref[:, lo:hi], (((0,), (0,)), ((), ())),
                            preferred_element_type=jnp.float32))
        wv[:, lo:hi] = w_col
        pt = pos[t:t + 1, :]
        acc = cums[t] * g_ref[:, lo:hi]
        for s in range(t):
            coef = jnp.where(pos[s:s + 1, :] == pt,
                             rows[t][s:s + 1, :], 0.0)
            acc = acc + coef * wv[:, s * B:(s + 1) * B]
        acc = acc + w_col
        out_ref[:, lo:hi] = (
            lax.dot_general(w_wr, w_col, (((0,), (0,)), ((), ())),
                            preferred_element_type=jnp.float32)
            + lax.dot_general(w_rd, acc, (((0,), (0,)), ((), ())),
                              preferred_element_type=jnp.float32)
            + bc)


def kernel(hidden, done, index, state0, W_write, W_pol, b_pol, W_val, b_val):
    index32 = index.astype(jnp.int32)

    idx2 = pl.pallas_call(
        _idx_kernel,
        out_shape=jax.ShapeDtypeStruct((TB, C), jnp.int32),
    )(index32).T

    st_flat = (state0.transpose(1, 2, 3, 0)
               .reshape(C * H * W // 8, 8, B // 128, 128)
               .transpose(0, 2, 1, 3)
               .reshape(B * C * H * W))

    g2 = pl.kernel(
        _sc_gather_body,
        out_type=jax.ShapeDtypeStruct((C, TB), jnp.float32),
        mesh=plsc.VectorSubcoreMesh(core_axis_name="c", subcore_axis_name="s",
                                    num_cores=NC, num_subcores=NS),
        scratch_types=[
            pltpu.VMEM((8, 2048), jnp.int32),
            pltpu.VMEM((8, 2048), jnp.float32),
            pltpu.SemaphoreType.DMA,
        ],
    )(st_flat, idx2)

    w_cat = jnp.concatenate([W_pol, W_val], axis=1)
    b_cat = jnp.concatenate([b_pol, b_val])
    pos2 = (index32[:, 0] * W + index32[:, 1]).reshape(T, B)
    done2 = done.reshape(T, B)

    resT = pl.pallas_call(
        _dense_kernel,
        out_shape=jax.ShapeDtypeStruct((N_OUT, TB), jnp.float32),
        scratch_shapes=[
            pltpu.VMEM((C, TB), jnp.float32),
        ],
    )(hidden.T, done2, pos2, g2, W_write, w_cat, b_cat)
    return resT.T


# speedup vs baseline: 125.8959x; 1.3728x over previous
import jax
import jax.numpy as jnp
from jax import lax
from jax.experimental import pallas as pl
from jax.experimental.pallas import tpu as pltpu, tpu_sc as plsc

T, B, C, H, W = 16, 1024, 32, 32, 32
D_IN = 160
N_ACT = 16
N_OUT = N_ACT + 1
TB = T * B

NC = 2
NS = 16
NW = NC * NS
CHUNK = 8


def _sc_gather_body(table_hbm, y_hbm, x_hbm, out_hbm, y_v, x_v, idx_v, rows_v,
                    sem):
    wid = lax.axis_index("s") * NC + lax.axis_index("c")
    c8 = wid // 8
    tbb = wid % 8
    rect = (pl.ds(c8 * 8, 8), pl.ds(tbb * 2048, 2048))
    pltpu.sync_copy(y_hbm.at[pl.ds(tbb * 2048, 2048)], y_v)
    pltpu.sync_copy(x_hbm.at[pl.ds(tbb * 2048, 2048)], x_v)

    def addr(jc, carry):
        j16 = jc * 16 + lax.broadcasted_iota(jnp.int32, (16,), 0)
        y16 = y_v[pl.ds(jc * 16, 16)]
        x16 = x_v[pl.ds(jc * 16, 16)]
        b = (tbb * 2048 + j16) & (B - 1)
        off = ((y16 * 4 + (x16 >> 3)) * 8192 + (b >> 7) * 1024
               + (x16 & 7) * 128 + (b & 127))
        base = off + (c8 * 8) * (H * W * B)
        for cs in range(8):
            idx_v[cs, pl.ds(jc * 16, 16)] = base + cs * (H * W * B)
        return carry

    lax.fori_loop(0, 2048 // 16, addr, 0)

    def chunk(jc, carry):
        base = jc * CHUNK
        for k in range(CHUNK):
            j = base + k
            cs = j % 8
            tl = (j // 8) * 128
            pltpu.make_async_copy(
                table_hbm.at[idx_v.at[cs, pl.ds(tl, 128)]],
                rows_v.at[cs, pl.ds(tl, 128)], sem,
            ).start()

        @pl.when(jc > 0)
        def _():
            for k in range(CHUNK):
                j = base - CHUNK + k
                cs = j % 8
                tl = (j // 8) * 128
                pltpu.make_async_copy(
                    table_hbm.at[pl.ds(0, 128)],
                    rows_v.at[cs, pl.ds(tl, 128)], sem,
                ).wait()

        return carry

    n_stream = 8 * 2048 // 128
    lax.fori_loop(0, n_stream // CHUNK, chunk, 0)
    for k in range(CHUNK):
        j = n_stream - CHUNK + k
        cs = j % 8
        tl = (j // 8) * 128
        pltpu.make_async_copy(
            table_hbm.at[pl.ds(0, 128)], rows_v.at[cs, pl.ds(tl, 128)], sem
        ).wait()
    pltpu.sync_copy(rows_v, out_hbm.at[rect])


def _dense_kernel(hT_ref, done_ref, pos_ref, g_ref, ww_ref, wc_ref, bc_ref,
                  out_ref, wv):
    pos = pos_ref[...]
    m = 1.0 - done_ref[...]

    e0 = lax.broadcasted_iota(jnp.int32, (T, B), 0)
    m0 = m[0:1, :]
    prev = jnp.where(e0 == 0, 1.0, 0.0)
    rows, cums = [prev], [m0]
    for t in range(1, T):
        mt = m[t:t + 1, :]
        prev = mt * prev + jnp.where(e0 == t, 1.0, 0.0)
        rows.append(prev)
        cums.append(prev[0:1, :] * m0)

    ww = ww_ref[...]
    w_wr = wc_ref[0:C, :]
    w_rd = wc_ref[C:2 * C, :]
    bc = bc_ref[...][:, None]

    for t in range(T):
        lo, hi = t * B, (t + 1) * B
        w_col = jnp.tanh(
            lax.dot_general(ww, hT_ref[:, lo:hi], (((0,), (0,)), ((), ())),
                            preferred_element_type=jnp.float32))
        wv[:, lo:hi] = w_col
        pt = pos[t:t + 1, :]
        acc = cums[t] * g_ref[:, lo:hi]
        for s in range(t):
            coef = jnp.where(pos[s:s + 1, :] == pt,
                             rows[t][s:s + 1, :], 0.0)
            acc = acc + coef * wv[:, s * B:(s + 1) * B]
        acc = acc + w_col
        out_ref[:, lo:hi] = (
            lax.dot_general(w_wr, w_col, (((0,), (0,)), ((), ())),
                            preferred_element_type=jnp.float32)
            + lax.dot_general(w_rd, acc, (((0,), (0,)), ((), ())),
                              preferred_element_type=jnp.float32)
            + bc)


def kernel(hidden, done, index, state0, W_write, W_pol, b_pol, W_val, b_val):
    index32 = index.astype(jnp.int32)

    st_flat = (state0.transpose(1, 2, 3, 0)
               .reshape(C * H * W // 8, 8, B // 128, 128)
               .transpose(0, 2, 1, 3)
               .reshape(B * C * H * W))

    g2 = pl.kernel(
        _sc_gather_body,
        out_type=jax.ShapeDtypeStruct((C, TB), jnp.float32),
        mesh=plsc.VectorSubcoreMesh(core_axis_name="c", subcore_axis_name="s",
                                    num_cores=NC, num_subcores=NS),
        scratch_types=[
            pltpu.VMEM((2048,), jnp.int32),
            pltpu.VMEM((2048,), jnp.int32),
            pltpu.VMEM((8, 2048), jnp.int32),
            pltpu.VMEM((8, 2048), jnp.float32),
            pltpu.SemaphoreType.DMA,
        ],
    )(st_flat, index32[:, 0], index32[:, 1])

    w_cat = jnp.concatenate([W_pol, W_val], axis=1)
    b_cat = jnp.concatenate([b_pol, b_val])
    pos2 = (index32[:, 0] * W + index32[:, 1]).reshape(T, B)
    done2 = done.reshape(T, B)

    resT = pl.pallas_call(
        _dense_kernel,
        out_shape=jax.ShapeDtypeStruct((N_OUT, TB), jnp.float32),
        scratch_shapes=[
            pltpu.VMEM((C, TB), jnp.float32),
        ],
    )(hidden.T, done2, pos2, g2, W_write, w_cat, b_cat)
    return resT.T


# speedup vs baseline: 129.4994x; 1.0286x over previous
import jax
import jax.numpy as jnp
from jax import lax
from jax.experimental import pallas as pl
from jax.experimental.pallas import tpu as pltpu, tpu_sc as plsc

T, B, C, H, W = 16, 1024, 32, 32, 32
D_IN = 160
N_ACT = 16
N_OUT = N_ACT + 1
TB = T * B

NC = 2
NS = 16
NW = NC * NS
CHUNK = 16


def _sc_gather_body(table_hbm, y_hbm, x_hbm, out_hbm, y_v, x_v, idx_v, rows_v,
                    sem):
    wid = lax.axis_index("s") * NC + lax.axis_index("c")
    c8 = wid // 8
    tbb = wid % 8
    rect = (pl.ds(c8 * 8, 8), pl.ds(tbb * 2048, 2048))
    pltpu.sync_copy(y_hbm.at[pl.ds(tbb * 2048, 2048)], y_v)
    pltpu.sync_copy(x_hbm.at[pl.ds(tbb * 2048, 2048)], x_v)

    def addr(jc, carry):
        j16 = jc * 16 + lax.broadcasted_iota(jnp.int32, (16,), 0)
        y16 = y_v[pl.ds(jc * 16, 16)]
        x16 = x_v[pl.ds(jc * 16, 16)]
        b = (tbb * 2048 + j16) & (B - 1)
        off = ((y16 * 4 + (x16 >> 3)) * 8192 + (b >> 7) * 1024
               + (x16 & 7) * 128 + (b & 127))
        base = off + (c8 * 8) * (H * W * B)
        for cs in range(8):
            idx_v[cs, pl.ds(jc * 16, 16)] = base + cs * (H * W * B)
        return carry

    lax.fori_loop(0, 2048 // 16, addr, 0)

    def chunk(jc, carry):
        base = jc * CHUNK
        for k in range(CHUNK):
            j = base + k
            cs = j % 8
            tl = (j // 8) * 128
            pltpu.make_async_copy(
                table_hbm.at[idx_v.at[cs, pl.ds(tl, 128)]],
                rows_v.at[cs, pl.ds(tl, 128)], sem,
            ).start()

        @pl.when(jc > 0)
        def _():
            for k in range(CHUNK):
                j = base - CHUNK + k
                cs = j % 8
                tl = (j // 8) * 128
                pltpu.make_async_copy(
                    table_hbm.at[pl.ds(0, 128)],
                    rows_v.at[cs, pl.ds(tl, 128)], sem,
                ).wait()

        return carry

    n_stream = 8 * 2048 // 128
    lax.fori_loop(0, n_stream // CHUNK, chunk, 0)
    for k in range(CHUNK):
        j = n_stream - CHUNK + k
        cs = j % 8
        tl = (j // 8) * 128
        pltpu.make_async_copy(
            table_hbm.at[pl.ds(0, 128)], rows_v.at[cs, pl.ds(tl, 128)], sem
        ).wait()
    pltpu.sync_copy(rows_v, out_hbm.at[rect])


def _dense_kernel(hT_ref, done_ref, pos_ref, g_ref, ww_ref, wc_ref, bc_ref,
                  out_ref, wv):
    pos = pos_ref[...]
    m = 1.0 - done_ref[...]

    e0 = lax.broadcasted_iota(jnp.int32, (T, B), 0)
    m0 = m[0:1, :]
    prev = jnp.where(e0 == 0, 1.0, 0.0)
    rows, cums = [prev], [m0]
    for t in range(1, T):
        mt = m[t:t + 1, :]
        prev = mt * prev + jnp.where(e0 == t, 1.0, 0.0)
        rows.append(prev)
        cums.append(prev[0:1, :] * m0)

    ww = ww_ref[...]
    w_wr = wc_ref[0:C, :]
    w_rd = wc_ref[C:2 * C, :]
    bc = bc_ref[...][:, None]

    for t in range(T):
        lo, hi = t * B, (t + 1) * B
        w_col = jnp.tanh(
            lax.dot_general(ww, hT_ref[:, lo:hi], (((0,), (0,)), ((), ())),
                            preferred_element_type=jnp.float32))
        wv[:, lo:hi] = w_col
        pt = pos[t:t + 1, :]
        acc = cums[t] * g_ref[:, lo:hi]
        for s in range(t):
            coef = jnp.where(pos[s:s + 1, :] == pt,
                             rows[t][s:s + 1, :], 0.0)
            acc = acc + coef * wv[:, s * B:(s + 1) * B]
        acc = acc + w_col
        out_ref[:, lo:hi] = (
            lax.dot_general(w_wr, w_col, (((0,), (0,)), ((), ())),
                            preferred_element_type=jnp.float32)
            + lax.dot_general(w_rd, acc, (((0,), (0,)), ((), ())),
                              preferred_element_type=jnp.float32)
            + bc)


def kernel(hidden, done, index, state0, W_write, W_pol, b_pol, W_val, b_val):
    index32 = index.astype(jnp.int32)

    st_flat = (state0.transpose(1, 2, 3, 0)
               .reshape(C * H * W // 8, 8, B // 128, 128)
               .transpose(0, 2, 1, 3)
               .reshape(B * C * H * W))

    g2 = pl.kernel(
        _sc_gather_body,
        out_type=jax.ShapeDtypeStruct((C, TB), jnp.float32),
        mesh=plsc.VectorSubcoreMesh(core_axis_name="c", subcore_axis_name="s",
                                    num_cores=NC, num_subcores=NS),
        scratch_types=[
            pltpu.VMEM((2048,), jnp.int32),
            pltpu.VMEM((2048,), jnp.int32),
            pltpu.VMEM((8, 2048), jnp.int32),
            pltpu.VMEM((8, 2048), jnp.float32),
            pltpu.SemaphoreType.DMA,
        ],
    )(st_flat, index32[:, 0], index32[:, 1])

    w_cat = jnp.concatenate([W_pol, W_val], axis=1)
    b_cat = jnp.concatenate([b_pol, b_val])
    pos2 = (index32[:, 0] * W + index32[:, 1]).reshape(T, B)
    done2 = done.reshape(T, B)

    resT = pl.pallas_call(
        _dense_kernel,
        out_shape=jax.ShapeDtypeStruct((N_OUT, TB), jnp.float32),
        scratch_shapes=[
            pltpu.VMEM((C, TB), jnp.float32),
        ],
    )(hidden.T, done2, pos2, g2, W_write, w_cat, b_cat)
    return resT.T
